# baseline (speedup 1.0000x reference)
import functools
import jax
import jax.numpy as jnp
from jax.experimental import pallas as pl
from jax.experimental.pallas import tpu as pltpu

IN_CH = 64
INTER_CH = 16
K_S = 100
SOFTMAX_SCALE = 10.0

Q_TOTAL = 2 * 64 * 64
QBLK = 512
NBLK = Q_TOTAL // QBLK


def _dists_body(x_ref, g_w_ref, g_b_ref, th_w_ref, th_b_ref, ph_w_ref,
                ph_b_ref, d_ref, v_ref):
    i = pl.program_id(0)
    x = x_ref[...]
    xb = x_ref[pl.ds(i * QBLK, QBLK), :]
    k = jnp.dot(x, ph_w_ref[...].T, preferred_element_type=jnp.float32)
    k = k + ph_b_ref[...][None, :]
    qb = jnp.dot(xb, g_w_ref[...].T, preferred_element_type=jnp.float32)
    qb = qb + g_b_ref[...][None, :]
    v = jnp.dot(x, th_w_ref[...].T, preferred_element_type=jnp.float32)
    v = v + th_b_ref[...][None, :]
    v_ref[...] = v
    kn = jnp.sum(k * k, axis=1)
    qn = jnp.sum(qb * qb, axis=1)
    qk = jnp.dot(qb, k.T, preferred_element_type=jnp.float32)
    d_ref[...] = -(qn[:, None] - 2.0 * qk + kn[None, :])


def _out_body(zi_ref, x_ref, W_w_ref, W_b_ref, y_ref):
    zi = zi_ref[...]
    y = jnp.dot(zi, W_w_ref[...].T, preferred_element_type=jnp.float32)
    y_ref[...] = y + W_b_ref[...][None, :] + x_ref[...]


def kernel(vid, g_w, g_b, theta_w, theta_b, phi_w, phi_b, W_w, W_b):
    t, c, h, w = vid.shape
    x = vid.transpose(0, 2, 3, 1).reshape(Q_TOTAL, IN_CH)

    dists, v = pl.pallas_call(
        _dists_body,
        grid=(NBLK,),
        in_specs=[
            pl.BlockSpec((Q_TOTAL, IN_CH), lambda i: (0, 0)),
            pl.BlockSpec((INTER_CH, IN_CH), lambda i: (0, 0)),
            pl.BlockSpec((INTER_CH,), lambda i: (0,)),
            pl.BlockSpec((INTER_CH, IN_CH), lambda i: (0, 0)),
            pl.BlockSpec((INTER_CH,), lambda i: (0,)),
            pl.BlockSpec((INTER_CH, IN_CH), lambda i: (0, 0)),
            pl.BlockSpec((INTER_CH,), lambda i: (0,)),
        ],
        out_specs=[
            pl.BlockSpec((QBLK, Q_TOTAL), lambda i: (i, 0)),
            pl.BlockSpec((Q_TOTAL, INTER_CH), lambda i: (0, 0)),
        ],
        out_shape=[
            jax.ShapeDtypeStruct((Q_TOTAL, Q_TOTAL), jnp.float32),
            jax.ShapeDtypeStruct((Q_TOTAL, INTER_CH), jnp.float32),
        ],
    )(x, g_w, g_b, theta_w, theta_b, phi_w, phi_b)

    topd, topi = jax.lax.top_k(dists, K_S)
    yi = jax.nn.softmax(topd * SOFTMAX_SCALE, axis=1)
    vg = jnp.take(v, topi, axis=0)
    zi = jnp.einsum('qk,qkd->qd', yi, vg)

    y = pl.pallas_call(
        _out_body,
        in_specs=[
            pl.BlockSpec((Q_TOTAL, INTER_CH), lambda: (0, 0)),
            pl.BlockSpec((Q_TOTAL, IN_CH), lambda: (0, 0)),
            pl.BlockSpec((IN_CH, INTER_CH), lambda: (0, 0)),
            pl.BlockSpec((IN_CH,), lambda: (0,)),
        ],
        out_specs=pl.BlockSpec((Q_TOTAL, IN_CH), lambda: (0, 0)),
        out_shape=jax.ShapeDtypeStruct((Q_TOTAL, IN_CH), jnp.float32),
    )(zi, x, W_w, W_b)

    y = y.reshape(t, h, w, c).transpose(0, 3, 1, 2)
    return (y, topi)


# speedup vs baseline: 7.4731x; 7.4731x over previous
import functools
import jax
import jax.numpy as jnp
from jax import lax
from jax.experimental import pallas as pl
from jax.experimental.pallas import tpu as pltpu, tpu_sc as plsc

IN_CH = 64
INTER_CH = 16
K_S = 100
SOFTMAX_SCALE = 10.0

Q = 8192
QBLK = 512
NBLK = Q // QBLK

KPAD = 112
NW = 32
ROWS_PER_W = Q // NW

def _dists_body(x_ref, g_w_ref, g_b_ref, th_w_ref, th_b_ref, ph_w_ref,
                ph_b_ref, key_ref, v_ref):
    i = pl.program_id(0)
    x = x_ref[...]
    xb = x_ref[pl.ds(i * QBLK, QBLK), :]
    k = jnp.dot(x, ph_w_ref[...].T, preferred_element_type=jnp.float32)
    k = k + ph_b_ref[...][None, :]
    qb = jnp.dot(xb, g_w_ref[...].T, preferred_element_type=jnp.float32)
    qb = qb + g_b_ref[...][None, :]
    v = jnp.dot(x, th_w_ref[...].T, preferred_element_type=jnp.float32)
    v_ref[:, :INTER_CH] = v + th_b_ref[...][None, :]
    kn = jnp.sum(k * k, axis=1)
    qn = jnp.sum(qb * qb, axis=1)
    qk = jnp.dot(qb, k.T, preferred_element_type=jnp.float32)
    d = -(qn[:, None] - 2.0 * qk + kn[None, :])
    u = lax.bitcast_convert_type(d, jnp.uint32)
    key = jnp.where(u >= jnp.uint32(0x80000000), ~u, u ^ jnp.uint32(0x80000000))
    key_ref[...] = lax.bitcast_convert_type(key, jnp.int32)


def _out_body(zi_ref, zs_ref, x_ref, W_w_ref, W_b_ref, y_ref):
    zi = zi_ref[...] / zs_ref[...]
    y = jnp.dot(zi, W_w_ref[...].T, preferred_element_type=jnp.float32)
    y_ref[...] = y + W_b_ref[...][None, :] + x_ref[...]


def _iota16():
    return jnp.arange(16, dtype=jnp.int32)


def _select_level(shift, n, need, a_cnt, inK, inI, outK, outI, hist, resK,
                  resI):
    nchunks = (n + 15) // 16

    @pl.loop(0, 16)
    def _zero(j):
        hist[pl.ds(j * 16, 16)] = jnp.zeros((16,), jnp.int32)

    @pl.loop(0, nchunks)
    def _hist(c):
        base = c * 16
        k16 = inK[pl.ds(base, 16)]
        valid = (_iota16() + base) < n
        digit = lax.shift_right_logical(k16, shift) & 0xFF
        plsc.addupdate_scatter(hist.at[:], [digit],
                               jnp.ones((16,), jnp.int32), mask=valid)

    def _scan(jj, carry):
        cum, found, bstar = carry
        j = 15 - jj
        h = hist[pl.ds(j * 16, 16)]
        cs = plsc.cumsum(lax.rev(h, (0,)))
        tot = cs + cum
        crossed = tot >= need
        cnt = jnp.sum(crossed.astype(jnp.int32))
        found_here = jnp.logical_and(cnt > 0, jnp.logical_not(found))
        bstar_new = j * 16 + cnt - 1
        bstar = jnp.where(found_here, bstar_new, bstar)
        found = jnp.logical_or(found, cnt > 0)
        cum = cum + jnp.max(cs)
        return cum, found, bstar

    _, _, bstar = lax.fori_loop(0, 16, _scan,
                                (jnp.int32(0), False, jnp.int32(0)))

    def _compact(c, carry):
        ac, ec = carry
        base = c * 16
        k16 = inK[pl.ds(base, 16)]
        valid = (_iota16() + base) < n
        if inI is None:
            i16 = _iota16() + base
        else:
            i16 = inI[pl.ds(base, 16)]
        digit = lax.shift_right_logical(k16, shift) & 0xFF
        m_above = jnp.logical_and(valid, digit > bstar)
        m_eq = jnp.logical_and(valid, digit == bstar)
        plsc.store_compressed(resK.at[pl.ds(ac, 16)], k16, mask=m_above)
        plsc.store_compressed(resI.at[pl.ds(ac, 16)], i16, mask=m_above)
        ac = ac + jnp.sum(m_above.astype(jnp.int32))
        plsc.store_compressed(outK.at[pl.ds(ec, 16)], k16, mask=m_eq)
        plsc.store_compressed(outI.at[pl.ds(ec, 16)], i16, mask=m_eq)
        ec = ec + jnp.sum(m_eq.astype(jnp.int32))
        return ac, ec

    return lax.fori_loop(0, nchunks, _compact, (a_cnt, jnp.int32(0)))


def _radix_pass(shift, srcK, srcI, dstK, dstI, hist, offs):
    @pl.loop(0, 16)
    def _zero(j):
        hist[pl.ds(j * 16, 16)] = jnp.zeros((16,), jnp.int32)

    for c in range(KPAD // 16):
        k16 = srcK[pl.ds(c * 16, 16)]
        digit = lax.shift_right_logical(~k16, shift) & 0xFF
        plsc.addupdate_scatter(hist.at[:], [digit],
                               jnp.ones((16,), jnp.int32), mask=None)

    def _prefix(j, cum):
        h = hist[pl.ds(j * 16, 16)]
        cs = plsc.cumsum(h)
        offs[pl.ds(j * 16, 16)] = cs - h + cum
        return cum + jnp.max(cs)

    lax.fori_loop(0, 16, _prefix, jnp.int32(0))

    for c in range(KPAD // 16):
        k16 = srcK[pl.ds(c * 16, 16)]
        i16 = srcI[pl.ds(c * 16, 16)]
        digit = lax.shift_right_logical(~k16, shift) & 0xFF
        occ, lastm = plsc.scan_count(digit)
        base = plsc.load_gather(offs.at[:], [digit])
        pos = base + occ - 1
        plsc.store_scatter(dstK.at[:], [pos], k16)
        plsc.store_scatter(dstI.at[:], [pos], i16)
        plsc.addupdate_scatter(offs.at[:], [digit], occ, mask=lastm)


def _sc_body(keys_hbm, v_hbm, topi_hbm, zi_hbm, zs_hbm, kbuf, eqKa, eqIa, eqKb, eqIb,
             hist, offs, resK, resI, tmpK, tmpI, wbuf, vbuf, zbuf, zsbuf, sem):
    cid = lax.axis_index('c')
    sid = lax.axis_index('s')
    wid = sid * 2 + cid
    row0 = wid * ROWS_PER_W

    @pl.loop(0, ROWS_PER_W)
    def _row(r):
        row = row0 + r
        pltpu.sync_copy(keys_hbm.at[pl.ds(row * Q, Q)], kbuf)

        ac = jnp.int32(0)
        need = jnp.int32(K_S)
        ac, ec = _select_level(24, jnp.int32(Q), need, ac, kbuf, None, eqKa,
                               eqIa, hist, resK, resI)
        need = jnp.int32(K_S) - ac
        ac, ec = _select_level(16, ec, need, ac, eqKa, eqIa, eqKb, eqIb,
                               hist, resK, resI)
        need = jnp.int32(K_S) - ac
        ac, ec = _select_level(8, ec, need, ac, eqKb, eqIb, eqKa, eqIa,
                               hist, resK, resI)
        need = jnp.int32(K_S) - ac
        ac, ec = _select_level(0, ec, need, ac, eqKa, eqIa, eqKb, eqIb,
                               hist, resK, resI)
        need = jnp.int32(K_S) - ac

        def _take(c, ac):
            base = c * 16
            valid = (_iota16() + base) < need
            k16 = eqKb[pl.ds(base, 16)]
            i16 = eqIb[pl.ds(base, 16)]
            plsc.store_compressed(resK.at[pl.ds(ac, 16)], k16, mask=valid)
            plsc.store_compressed(resI.at[pl.ds(ac, 16)], i16, mask=valid)
            return ac + jnp.sum(valid.astype(jnp.int32))

        ac = lax.fori_loop(0, (need + 15) // 16, _take, ac)

        padm = _iota16() < 4
        resK[pl.ds(96, 16)] = jnp.where(padm, resK[pl.ds(96, 16)],
                                        jnp.int32(0))
        resI[pl.ds(96, 16)] = jnp.where(padm, resI[pl.ds(96, 16)],
                                        jnp.int32(0))

        _radix_pass(0, resK, resI, tmpK, tmpI, hist, offs)
        _radix_pass(8, tmpK, tmpI, resK, resI, hist, offs)
        _radix_pass(16, resK, resI, tmpK, tmpI, hist, offs)
        _radix_pass(24, tmpK, tmpI, resK, resI, hist, offs)

        def _dist_chunk(c):
            k16 = resK[pl.ds(c * 16, 16)]
            s = jnp.where(k16 < 0, k16 ^ jnp.int32(-0x80000000), ~k16)
            d = lax.bitcast_convert_type(s, jnp.float32)
            return d * jnp.float32(SOFTMAX_SCALE)

        m = jnp.float32(-3.0e38)
        for c in range(KPAD // 16):
            x = _dist_chunk(c)
            if c == KPAD // 16 - 1:
                x = jnp.where(_iota16() < 4, x, jnp.float32(-3.0e38))
            wbuf[pl.ds(c * 16, 16)] = x
            m = jnp.maximum(m, jnp.max(x))

        ssum = jnp.float32(0.0)
        for c in range(KPAD // 16):
            x = wbuf[pl.ds(c * 16, 16)]
            e = jnp.exp(x - m)
            if c == KPAD // 16 - 1:
                e = jnp.where(_iota16() < 4, e, jnp.float32(0.0))
            wbuf[pl.ds(c * 16, 16)] = e
            ssum = ssum + jnp.sum(e)

        pltpu.async_copy(v_hbm.at[resI.at[pl.ds(0, KPAD)]], vbuf,
                         sem).wait()

        acc = jnp.zeros((16,), jnp.float32)
        for c in range(KPAD // 16):
            wk = wbuf[pl.ds(c * 16, 16)]
            for jj in range(16):
                acc = acc + vbuf[c * 16 + jj, :INTER_CH] * wk[jj]
        zbuf[...] = acc
        zsbuf[...] = jnp.zeros((16,), jnp.float32) + ssum

        pltpu.sync_copy(resI.at[pl.ds(0, KPAD)],
                        topi_hbm.at[pl.ds(row * KPAD, KPAD)])
        pltpu.sync_copy(zbuf, zi_hbm.at[pl.ds(row * 16, 16)])
        pltpu.sync_copy(zsbuf, zs_hbm.at[pl.ds(row * 16, 16)])


def kernel(vid, g_w, g_b, theta_w, theta_b, phi_w, phi_b, W_w, W_b):
    t, c, h, w = vid.shape
    x = vid.transpose(0, 2, 3, 1).reshape(Q, IN_CH)

    keys, v = pl.pallas_call(
        _dists_body,
        grid=(NBLK,),
        in_specs=[
            pl.BlockSpec((Q, IN_CH), lambda i: (0, 0)),
            pl.BlockSpec((INTER_CH, IN_CH), lambda i: (0, 0)),
            pl.BlockSpec((INTER_CH,), lambda i: (0,)),
            pl.BlockSpec((INTER_CH, IN_CH), lambda i: (0, 0)),
            pl.BlockSpec((INTER_CH,), lambda i: (0,)),
            pl.BlockSpec((INTER_CH, IN_CH), lambda i: (0, 0)),
            pl.BlockSpec((INTER_CH,), lambda i: (0,)),
        ],
        out_specs=[
            pl.BlockSpec((QBLK, Q), lambda i: (i, 0)),
            pl.BlockSpec((Q, 128), lambda i: (0, 0)),
        ],
        out_shape=[
            jax.ShapeDtypeStruct((Q, Q), jnp.int32),
            jax.ShapeDtypeStruct((Q, 128), jnp.float32),
        ],
    )(x, g_w, g_b, theta_w, theta_b, phi_w, phi_b)

    sc = pl.kernel(
        _sc_body,
        out_type=[
            jax.ShapeDtypeStruct((Q * KPAD,), jnp.int32),
            jax.ShapeDtypeStruct((Q * 16,), jnp.float32),
            jax.ShapeDtypeStruct((Q * 16,), jnp.float32),
        ],
        mesh=plsc.VectorSubcoreMesh(core_axis_name='c', subcore_axis_name='s'),
        scratch_types=[
            pltpu.VMEM((Q,), jnp.int32),
            pltpu.VMEM((Q + 16,), jnp.int32),
            pltpu.VMEM((Q + 16,), jnp.int32),
            pltpu.VMEM((Q + 16,), jnp.int32),
            pltpu.VMEM((Q + 16,), jnp.int32),
            pltpu.VMEM((256,), jnp.int32),
            pltpu.VMEM((256,), jnp.int32),
            pltpu.VMEM((KPAD + 16,), jnp.int32),
            pltpu.VMEM((KPAD + 16,), jnp.int32),
            pltpu.VMEM((KPAD,), jnp.int32),
            pltpu.VMEM((KPAD,), jnp.int32),
            pltpu.VMEM((KPAD,), jnp.float32),
            pltpu.VMEM((KPAD, 128), jnp.float32),
            pltpu.VMEM((16,), jnp.float32),
            pltpu.VMEM((16,), jnp.float32),
            pltpu.SemaphoreType.DMA,
        ],
        compiler_params=pltpu.CompilerParams(needs_layout_passes=False),
    )
    topi_pad, zi, zs = sc(keys.reshape(Q * Q), v)
    topi = topi_pad.reshape(Q, KPAD)[:, :K_S]
    zi = zi.reshape(Q, INTER_CH)
    zs = zs.reshape(Q, INTER_CH)

    y = pl.pallas_call(
        _out_body,
        in_specs=[
            pl.BlockSpec((Q, INTER_CH), lambda: (0, 0)),
            pl.BlockSpec((Q, INTER_CH), lambda: (0, 0)),
            pl.BlockSpec((Q, IN_CH), lambda: (0, 0)),
            pl.BlockSpec((IN_CH, INTER_CH), lambda: (0, 0)),
            pl.BlockSpec((IN_CH,), lambda: (0,)),
        ],
        out_specs=pl.BlockSpec((Q, IN_CH), lambda: (0, 0)),
        out_shape=jax.ShapeDtypeStruct((Q, IN_CH), jnp.float32),
    )(zi, zs, x, W_w, W_b)

    y = y.reshape(t, h, w, c).transpose(0, 3, 1, 2)
    return (y, topi)


# speedup vs baseline: 7.5188x; 1.0061x over previous
import functools
import jax
import jax.numpy as jnp
from jax import lax
from jax.experimental import pallas as pl
from jax.experimental.pallas import tpu as pltpu, tpu_sc as plsc

IN_CH = 64
INTER_CH = 16
K_S = 100
SOFTMAX_SCALE = 10.0

Q = 8192
QBLK = 512
NBLK = Q // QBLK

KPAD = 112
NW = 32
ROWS_PER_W = Q // NW

def _dists_body(x_ref, g_w_ref, g_b_ref, th_w_ref, th_b_ref, ph_w_ref,
                ph_b_ref, key_ref, v_ref):
    i = pl.program_id(0)
    x = x_ref[...]
    xb = x_ref[pl.ds(i * QBLK, QBLK), :]
    k = jnp.dot(x, ph_w_ref[...].T, preferred_element_type=jnp.float32)
    k = k + ph_b_ref[...][None, :]
    qb = jnp.dot(xb, g_w_ref[...].T, preferred_element_type=jnp.float32)
    qb = qb + g_b_ref[...][None, :]
    v = jnp.dot(x, th_w_ref[...].T, preferred_element_type=jnp.float32)
    v_ref[:, :INTER_CH] = v + th_b_ref[...][None, :]
    kn = jnp.sum(k * k, axis=1)
    qn = jnp.sum(qb * qb, axis=1)
    qk = jnp.dot(qb, k.T, preferred_element_type=jnp.float32)
    d = -(qn[:, None] - 2.0 * qk + kn[None, :])
    u = lax.bitcast_convert_type(d, jnp.uint32)
    key = jnp.where(u >= jnp.uint32(0x80000000), ~u, u ^ jnp.uint32(0x80000000))
    key_ref[...] = lax.bitcast_convert_type(key, jnp.int32)


def _out_body(zi_ref, zs_ref, x_ref, W_w_ref, W_b_ref, y_ref):
    zi = zi_ref[...] / zs_ref[...]
    y = jnp.dot(zi, W_w_ref[...].T, preferred_element_type=jnp.float32)
    y_ref[...] = y + W_b_ref[...][None, :] + x_ref[...]


def _iota16():
    return jnp.arange(16, dtype=jnp.int32)


def _select_level(shift, n, need, a_cnt, inK, inI, outK, outI, hist, resK,
                  resI):
    static_n = isinstance(n, int)
    nchunks = (n + 15) // 16

    @pl.loop(0, 16, unroll=4)
    def _zero(j):
        hist[pl.ds(j * 16, 16)] = jnp.zeros((16,), jnp.int32)

    def _hist(c):
        base = c * 16
        k16 = inK[pl.ds(base, 16)]
        valid = None if static_n else (_iota16() + base) < n
        digit = lax.shift_right_logical(k16, shift) & 0xFF
        plsc.addupdate_scatter(hist.at[:], [digit],
                               jnp.ones((16,), jnp.int32), mask=valid)

    if static_n:
        pl.loop(0, nchunks, unroll=8)(_hist)
    else:
        pl.loop(0, nchunks)(_hist)

    def _scan(jj, carry):
        cum, found, bstar = carry
        j = 15 - jj
        h = hist[pl.ds(j * 16, 16)]
        cs = plsc.cumsum(lax.rev(h, (0,)))
        tot = cs + cum
        crossed = tot >= need
        cnt = plsc.all_reduce_population_count(crossed)[0]
        found_here = jnp.logical_and(cnt > 0, jnp.logical_not(found))
        bstar_new = j * 16 + cnt - 1
        bstar = jnp.where(found_here, bstar_new, bstar)
        found = jnp.logical_or(found, cnt > 0)
        cum = cum + cs[15]
        return cum, found, bstar

    _, _, bstar = lax.fori_loop(0, 16, _scan,
                                (jnp.int32(0), False, jnp.int32(0)),
                                unroll=2)

    def _compact(c, carry):
        ac, ec = carry
        base = c * 16
        k16 = inK[pl.ds(base, 16)]
        if inI is None:
            i16 = _iota16() + base
        else:
            i16 = inI[pl.ds(base, 16)]
        digit = lax.shift_right_logical(k16, shift) & 0xFF
        m_above = digit > bstar
        m_eq = digit == bstar
        if not static_n:
            valid = (_iota16() + base) < n
            m_above = jnp.logical_and(valid, m_above)
            m_eq = jnp.logical_and(valid, m_eq)
        plsc.store_compressed(resK.at[pl.ds(ac, 16)], k16, mask=m_above)
        plsc.store_compressed(resI.at[pl.ds(ac, 16)], i16, mask=m_above)
        ac = ac + plsc.all_reduce_population_count(m_above)[0]
        plsc.store_compressed(outK.at[pl.ds(ec, 16)], k16, mask=m_eq)
        plsc.store_compressed(outI.at[pl.ds(ec, 16)], i16, mask=m_eq)
        ec = ec + plsc.all_reduce_population_count(m_eq)[0]
        return ac, ec

    if static_n:
        return lax.fori_loop(0, nchunks, _compact, (a_cnt, jnp.int32(0)),
                             unroll=4)
    return lax.fori_loop(0, nchunks, _compact, (a_cnt, jnp.int32(0)))


def _radix_pass(shift, srcK, srcI, dstK, dstI, hist, offs):
    @pl.loop(0, 16, unroll=4)
    def _zero(j):
        hist[pl.ds(j * 16, 16)] = jnp.zeros((16,), jnp.int32)

    for c in range(KPAD // 16):
        k16 = srcK[pl.ds(c * 16, 16)]
        digit = lax.shift_right_logical(~k16, shift) & 0xFF
        plsc.addupdate_scatter(hist.at[:], [digit],
                               jnp.ones((16,), jnp.int32), mask=None)

    def _prefix(j, cum):
        h = hist[pl.ds(j * 16, 16)]
        cs = plsc.cumsum(h)
        offs[pl.ds(j * 16, 16)] = cs - h + cum
        return cum + cs[15]

    lax.fori_loop(0, 16, _prefix, jnp.int32(0), unroll=2)

    for c in range(KPAD // 16):
        k16 = srcK[pl.ds(c * 16, 16)]
        i16 = srcI[pl.ds(c * 16, 16)]
        digit = lax.shift_right_logical(~k16, shift) & 0xFF
        occ, lastm = plsc.scan_count(digit)
        base = plsc.load_gather(offs.at[:], [digit])
        pos = base + occ - 1
        plsc.store_scatter(dstK.at[:], [pos], k16)
        plsc.store_scatter(dstI.at[:], [pos], i16)
        plsc.addupdate_scatter(offs.at[:], [digit], occ, mask=lastm)


def _sc_body(keys_hbm, v_hbm, topi_hbm, zi_hbm, zs_hbm, kbuf, eqKa, eqIa, eqKb, eqIb,
             hist, offs, resK, resI, tmpK, tmpI, wbuf, vbuf, zbuf, zsbuf, sem):
    cid = lax.axis_index('c')
    sid = lax.axis_index('s')
    wid = sid * 2 + cid
    row0 = wid * ROWS_PER_W

    @pl.loop(0, ROWS_PER_W)
    def _row(r):
        row = row0 + r
        pltpu.sync_copy(keys_hbm.at[pl.ds(row * Q, Q)], kbuf)

        ac = jnp.int32(0)
        need = jnp.int32(K_S)
        ac, ec = _select_level(24, Q, need, ac, kbuf, None, eqKa,
                               eqIa, hist, resK, resI)
        need = jnp.int32(K_S) - ac
        ac, ec = _select_level(16, ec, need, ac, eqKa, eqIa, eqKb, eqIb,
                               hist, resK, resI)
        need = jnp.int32(K_S) - ac
        ac, ec = _select_level(8, ec, need, ac, eqKb, eqIb, eqKa, eqIa,
                               hist, resK, resI)
        need = jnp.int32(K_S) - ac
        ac, ec = _select_level(0, ec, need, ac, eqKa, eqIa, eqKb, eqIb,
                               hist, resK, resI)
        need = jnp.int32(K_S) - ac

        def _take(c, ac):
            base = c * 16
            valid = (_iota16() + base) < need
            k16 = eqKb[pl.ds(base, 16)]
            i16 = eqIb[pl.ds(base, 16)]
            plsc.store_compressed(resK.at[pl.ds(ac, 16)], k16, mask=valid)
            plsc.store_compressed(resI.at[pl.ds(ac, 16)], i16, mask=valid)
            return ac + plsc.all_reduce_population_count(valid)[0]

        ac = lax.fori_loop(0, (need + 15) // 16, _take, ac)

        padm = _iota16() < 4
        resK[pl.ds(96, 16)] = jnp.where(padm, resK[pl.ds(96, 16)],
                                        jnp.int32(0))
        resI[pl.ds(96, 16)] = jnp.where(padm, resI[pl.ds(96, 16)],
                                        jnp.int32(0))

        _radix_pass(0, resK, resI, tmpK, tmpI, hist, offs)
        _radix_pass(8, tmpK, tmpI, resK, resI, hist, offs)
        _radix_pass(16, resK, resI, tmpK, tmpI, hist, offs)
        _radix_pass(24, tmpK, tmpI, resK, resI, hist, offs)

        def _dist_chunk(c):
            k16 = resK[pl.ds(c * 16, 16)]
            s = jnp.where(k16 < 0, k16 ^ jnp.int32(-0x80000000), ~k16)
            d = lax.bitcast_convert_type(s, jnp.float32)
            return d * jnp.float32(SOFTMAX_SCALE)

        m = jnp.float32(-3.0e38)
        for c in range(KPAD // 16):
            x = _dist_chunk(c)
            if c == KPAD // 16 - 1:
                x = jnp.where(_iota16() < 4, x, jnp.float32(-3.0e38))
            wbuf[pl.ds(c * 16, 16)] = x
            m = jnp.maximum(m, jnp.max(x))

        ssum = jnp.float32(0.0)
        for c in range(KPAD // 16):
            x = wbuf[pl.ds(c * 16, 16)]
            e = jnp.exp(x - m)
            if c == KPAD // 16 - 1:
                e = jnp.where(_iota16() < 4, e, jnp.float32(0.0))
            wbuf[pl.ds(c * 16, 16)] = e
            ssum = ssum + jnp.sum(e)

        pltpu.async_copy(v_hbm.at[resI.at[pl.ds(0, KPAD)]], vbuf,
                         sem).wait()

        acc = jnp.zeros((16,), jnp.float32)
        for c in range(KPAD // 16):
            wk = wbuf[pl.ds(c * 16, 16)]
            for jj in range(16):
                acc = acc + vbuf[c * 16 + jj, :INTER_CH] * wk[jj]
        zbuf[...] = acc
        zsbuf[...] = jnp.zeros((16,), jnp.float32) + ssum

        pltpu.sync_copy(resI.at[pl.ds(0, KPAD)],
                        topi_hbm.at[pl.ds(row * KPAD, KPAD)])
        pltpu.sync_copy(zbuf, zi_hbm.at[pl.ds(row * 16, 16)])
        pltpu.sync_copy(zsbuf, zs_hbm.at[pl.ds(row * 16, 16)])


def kernel(vid, g_w, g_b, theta_w, theta_b, phi_w, phi_b, W_w, W_b):
    t, c, h, w = vid.shape
    x = vid.transpose(0, 2, 3, 1).reshape(Q, IN_CH)

    keys, v = pl.pallas_call(
        _dists_body,
        grid=(NBLK,),
        in_specs=[
            pl.BlockSpec((Q, IN_CH), lambda i: (0, 0)),
            pl.BlockSpec((INTER_CH, IN_CH), lambda i: (0, 0)),
            pl.BlockSpec((INTER_CH,), lambda i: (0,)),
            pl.BlockSpec((INTER_CH, IN_CH), lambda i: (0, 0)),
            pl.BlockSpec((INTER_CH,), lambda i: (0,)),
            pl.BlockSpec((INTER_CH, IN_CH), lambda i: (0, 0)),
            pl.BlockSpec((INTER_CH,), lambda i: (0,)),
        ],
        out_specs=[
            pl.BlockSpec((QBLK, Q), lambda i: (i, 0)),
            pl.BlockSpec((Q, 128), lambda i: (0, 0)),
        ],
        out_shape=[
            jax.ShapeDtypeStruct((Q, Q), jnp.int32),
            jax.ShapeDtypeStruct((Q, 128), jnp.float32),
        ],
    )(x, g_w, g_b, theta_w, theta_b, phi_w, phi_b)

    sc = pl.kernel(
        _sc_body,
        out_type=[
            jax.ShapeDtypeStruct((Q * KPAD,), jnp.int32),
            jax.ShapeDtypeStruct((Q * 16,), jnp.float32),
            jax.ShapeDtypeStruct((Q * 16,), jnp.float32),
        ],
        mesh=plsc.VectorSubcoreMesh(core_axis_name='c', subcore_axis_name='s'),
        scratch_types=[
            pltpu.VMEM((Q,), jnp.int32),
            pltpu.VMEM((Q + 16,), jnp.int32),
            pltpu.VMEM((Q + 16,), jnp.int32),
            pltpu.VMEM((Q + 16,), jnp.int32),
            pltpu.VMEM((Q + 16,), jnp.int32),
            pltpu.VMEM((256,), jnp.int32),
            pltpu.VMEM((256,), jnp.int32),
            pltpu.VMEM((KPAD + 16,), jnp.int32),
            pltpu.VMEM((KPAD + 16,), jnp.int32),
            pltpu.VMEM((KPAD,), jnp.int32),
            pltpu.VMEM((KPAD,), jnp.int32),
            pltpu.VMEM((KPAD,), jnp.float32),
            pltpu.VMEM((KPAD, 128), jnp.float32),
            pltpu.VMEM((16,), jnp.float32),
            pltpu.VMEM((16,), jnp.float32),
            pltpu.SemaphoreType.DMA,
        ],
        compiler_params=pltpu.CompilerParams(needs_layout_passes=False),
    )
    topi_pad, zi, zs = sc(keys.reshape(Q * Q), v)
    topi = topi_pad.reshape(Q, KPAD)[:, :K_S]
    zi = zi.reshape(Q, INTER_CH)
    zs = zs.reshape(Q, INTER_CH)

    y = pl.pallas_call(
        _out_body,
        in_specs=[
            pl.BlockSpec((Q, INTER_CH), lambda: (0, 0)),
            pl.BlockSpec((Q, INTER_CH), lambda: (0, 0)),
            pl.BlockSpec((Q, IN_CH), lambda: (0, 0)),
            pl.BlockSpec((IN_CH, INTER_CH), lambda: (0, 0)),
            pl.BlockSpec((IN_CH,), lambda: (0,)),
        ],
        out_specs=pl.BlockSpec((Q, IN_CH), lambda: (0, 0)),
        out_shape=jax.ShapeDtypeStruct((Q, IN_CH), jnp.float32),
    )(zi, zs, x, W_w, W_b)

    y = y.reshape(t, h, w, c).transpose(0, 3, 1, 2)
    return (y, topi)
